# baseline (speedup 1.0000x reference)
import functools

import jax
import jax.numpy as jnp
from jax import lax
from jax.experimental import pallas as pl
from jax.experimental.pallas import tpu as pltpu
from jax.experimental.pallas import tpu_sc as plsc

N = 10000
E = 320000
D = 128

NC = 2
NS = 16
NW = NC * NS

ROWS_PER_TILE = E // NW
CHUNK = 80
BIG = 2 * CHUNK
NBIG = ROWS_PER_TILE // BIG
WIN = 640
WIN_STRIDE = 624


def _sc_partials(src, index):
    mesh = plsc.VectorSubcoreMesh(core_axis_name="c", subcore_axis_name="s")

    @functools.partial(
        pl.kernel,
        mesh=mesh,
        out_type=jax.ShapeDtypeStruct((NC, N, D), jnp.float32),
        scratch_types=[
            pltpu.VMEM_SHARED((N, D), jnp.float32),
            pltpu.VMEM((4, CHUNK), jnp.int32),
            pltpu.VMEM((16, D), jnp.float32),
            pltpu.VMEM((BIG, D), jnp.float32),
            pltpu.VMEM((BIG, D), jnp.float32),
        ]
        + [pltpu.SemaphoreType.DMA for _ in range(7)],
    )
    def body(src_hbm, idx_hbm, out_hbm, acc, idx_r, zbuf, buf0, buf1, *sems):
        bufs = (buf0, buf1)
        fsems = sems[0:2]
        isems = sems[2:4]
        ssems = sems[4:6]
        zsem = sems[6]
        c = lax.axis_index("c")
        s = lax.axis_index("s")
        wid = c * NS + s
        row0 = wid * ROWS_PER_TILE
        win0 = pl.multiple_of(s * WIN_STRIDE, 8)

        def fill_start(j, b):
            src_off = pl.multiple_of(row0 + j * BIG, 8)
            pltpu.async_copy(
                idx_hbm.at[pl.ds(src_off, CHUNK)], idx_r.at[2 * b], isems[b]
            )
            pltpu.async_copy(
                idx_hbm.at[pl.ds(pl.multiple_of(src_off + CHUNK, 8), CHUNK)],
                idx_r.at[2 * b + 1],
                isems[b],
            )
            pltpu.async_copy(src_hbm.at[pl.ds(src_off, BIG)], bufs[b], fsems[b])

        def fill_wait(b):
            pltpu.make_async_copy(
                src_hbm.at[pl.ds(0, BIG)], bufs[b], fsems[b]
            ).wait()
            for _ in range(2):
                pltpu.make_async_copy(
                    idx_hbm.at[pl.ds(0, CHUNK)], idx_r.at[2 * b], isems[b]
                ).wait()

        def scat_start(b):
            for h in range(2):
                pltpu.async_copy(
                    bufs[b].at[pl.ds(h * CHUNK, CHUNK)],
                    acc.at[idx_r.at[2 * b + h]],
                    ssems[b],
                    add=True,
                )

        def scat_wait(b):
            for h in range(2):
                pltpu.make_async_copy(
                    bufs[b].at[pl.ds(0, CHUNK)], acc.at[idx_r.at[2 * b]], ssems[b]
                ).wait()

        def tail_start():
            src_off = pl.multiple_of(row0 + NBIG * BIG, 8)
            pltpu.async_copy(
                idx_hbm.at[pl.ds(src_off, CHUNK)], idx_r.at[0], isems[0]
            )
            pltpu.async_copy(
                src_hbm.at[pl.ds(src_off, CHUNK)],
                bufs[0].at[pl.ds(0, CHUNK)],
                fsems[0],
            )

        def tail_wait():
            pltpu.make_async_copy(
                src_hbm.at[pl.ds(0, CHUNK)], bufs[0].at[pl.ds(0, CHUNK)], fsems[0]
            ).wait()
            pltpu.make_async_copy(
                idx_hbm.at[pl.ds(0, CHUNK)], idx_r.at[0], isems[0]
            ).wait()

        fill_start(0, 0)
        fill_start(1, 1)

        zeros16 = jnp.zeros((16,), jnp.float32)

        def zero_row(r, _):
            for k in range(D // 16):
                zbuf[r, pl.ds(k * 16, 16)] = zeros16
            return 0

        lax.fori_loop(0, 16, zero_row, 0)
        for z in range(WIN // 16):
            pltpu.async_copy(
                zbuf,
                acc.at[pl.ds(pl.multiple_of(win0 + z * 16, 8), 16)],
                zsem,
            )
        for z in range(WIN // 16):
            pltpu.make_async_copy(zbuf, acc.at[pl.ds(win0, 16)], zsem).wait()
        plsc.subcore_barrier()

        fill_wait(0)
        scat_start(0)

        def pair_step(g, _):
            fill_wait(1)
            scat_start(1)
            scat_wait(0)
            fill_start(2 * g + 2, 0)
            fill_wait(0)
            scat_start(0)
            scat_wait(1)
            fill_start(2 * g + 3, 1)
            return 0

        lax.fori_loop(0, (NBIG - 2) // 2, pair_step, 0)

        fill_wait(1)
        scat_start(1)
        scat_wait(0)
        tail_start()
        tail_wait()
        pltpu.async_copy(
            bufs[0].at[pl.ds(0, CHUNK)], acc.at[idx_r.at[0]], ssems[0], add=True
        )
        scat_wait(1)
        pltpu.make_async_copy(
            bufs[0].at[pl.ds(0, CHUNK)], acc.at[idx_r.at[0]], ssems[0]
        ).wait()
        plsc.subcore_barrier()

        pltpu.sync_copy(
            acc.at[pl.ds(win0, WIN)],
            out_hbm.at[c].at[pl.ds(win0, WIN)],
        )

    return body(src, index)


def _combine(partials):
    def body(p_ref, o_ref):
        o_ref[...] = p_ref[0] + p_ref[1]

    blk = 1000
    return pl.pallas_call(
        body,
        grid=(N // blk,),
        in_specs=[pl.BlockSpec((NC, blk, D), lambda i: (0, i, 0))],
        out_specs=pl.BlockSpec((blk, D), lambda i: (i, 0)),
        out_shape=jax.ShapeDtypeStruct((N, D), jnp.float32),
    )(partials)


def kernel(src, index):
    partials = _sc_partials(src, index)
    return _combine(partials)
